# Initial kernel scaffold; baseline (speedup 1.0000x reference)
import jax
import jax.numpy as jnp
from jax.experimental import pallas as pl


def kernel(input, in_weight, in_bias, out_weight):
    raise NotImplementedError("write your pallas kernel here")


# speedup vs baseline: 1.7350x; 1.7350x over previous
import functools
import math

import jax
import jax.numpy as jnp
from jax.experimental import pallas as pl
from jax.experimental.pallas import tpu as pltpu

_DEPTH = 11
_NLEVELS = _DEPTH + 1
_N_NODES = 2 ** _NLEVELS - 1
_WIDTH = 2048
_TILE = 256
_LVL_PAD = 128


def _path_kernel(x_ref, w_ref, b_ref, acts_ref, nodes_ref):
    x = x_ref[...]
    t = x.shape[0]
    n = jnp.zeros((t, 1), jnp.int32)
    acts = jnp.zeros((t, _LVL_PAD), jnp.float32)
    nodes = jnp.zeros((t, _LVL_PAD), jnp.int32)
    lane = jax.lax.broadcasted_iota(jnp.int32, (t, _LVL_PAD), 1)
    for d in range(_NLEVELS):
        start = 2 ** d - 1
        size = 2 ** d
        w = w_ref[start:start + size, :]
        logits = jax.lax.dot_general(
            x, w, (((1,), (1,)), ((), ())),
            precision=jax.lax.Precision.HIGHEST,
            preferred_element_type=jnp.float32)
        local = n - start
        ids = jax.lax.broadcasted_iota(jnp.int32, (t, size), 1)
        sel = ids == local
        logit = jnp.sum(jnp.where(sel, logits, 0.0), axis=1, keepdims=True)
        bias = b_ref[d:d + 1, :size]
        bsel = jnp.sum(jnp.where(sel, jnp.broadcast_to(bias, (t, size)), 0.0),
                       axis=1, keepdims=True)
        logit = logit + bsel
        act = 0.5 * logit * (1.0 + jax.lax.erf(logit * (1.0 / math.sqrt(2.0))))
        acts = jnp.where(lane == d, act, acts)
        nodes = jnp.where(lane == d, n, nodes)
        n = 2 * n + 1 + (logit > 0.0).astype(jnp.int32)
    acts_ref[...] = acts
    nodes_ref[...] = nodes


def _out_kernel(acts_ref, nodes_ref, wout_ref, out_ref):
    acts = acts_ref[...]
    nodes = nodes_ref[...]
    t = acts.shape[0]
    cols = jax.lax.broadcasted_iota(jnp.int32, (t, _N_NODES + 1), 1)
    a = jnp.zeros((t, _N_NODES + 1), jnp.float32)
    for d in range(_NLEVELS):
        nd = nodes[:, d:d + 1]
        ad = acts[:, d:d + 1]
        a = jnp.where(cols == nd, ad, a)
    out_ref[...] = jax.lax.dot_general(
        a.astype(jnp.bfloat16), wout_ref[...],
        (((1,), (0,)), ((), ())),
        preferred_element_type=jnp.float32)


def kernel(input, in_weight, in_bias, out_weight):
    orig_shape = input.shape
    x = input.reshape(-1, _WIDTH)
    tokens = x.shape[0]
    ntiles = tokens // _TILE

    w_in = jnp.pad(in_weight, ((0, 1), (0, 0)))
    w_out = jnp.pad(out_weight, ((0, 1), (0, 0))).astype(jnp.bfloat16)
    bias_lvl = jnp.zeros((_NLEVELS + 4, _WIDTH), jnp.float32)
    for d in range(_NLEVELS):
        row = jnp.zeros((_WIDTH,), jnp.float32)
        row = jax.lax.dynamic_update_slice(
            row, in_bias[2 ** d - 1: 2 ** (d + 1) - 1], (0,))
        bias_lvl = bias_lvl.at[d].set(row)

    acts, nodes = pl.pallas_call(
        _path_kernel,
        grid=(ntiles,),
        in_specs=[
            pl.BlockSpec((_TILE, _WIDTH), lambda i: (i, 0)),
            pl.BlockSpec((_N_NODES + 1, _WIDTH), lambda i: (0, 0)),
            pl.BlockSpec((_NLEVELS + 4, _WIDTH), lambda i: (0, 0)),
        ],
        out_specs=[
            pl.BlockSpec((_TILE, _LVL_PAD), lambda i: (i, 0)),
            pl.BlockSpec((_TILE, _LVL_PAD), lambda i: (i, 0)),
        ],
        out_shape=[
            jax.ShapeDtypeStruct((tokens, _LVL_PAD), jnp.float32),
            jax.ShapeDtypeStruct((tokens, _LVL_PAD), jnp.int32),
        ],
    )(x, w_in, bias_lvl)

    out = pl.pallas_call(
        _out_kernel,
        grid=(ntiles,),
        in_specs=[
            pl.BlockSpec((_TILE, _LVL_PAD), lambda i: (i, 0)),
            pl.BlockSpec((_TILE, _LVL_PAD), lambda i: (i, 0)),
            pl.BlockSpec((_N_NODES + 1, _WIDTH), lambda i: (0, 0)),
        ],
        out_specs=pl.BlockSpec((_TILE, _WIDTH), lambda i: (i, 0)),
        out_shape=jax.ShapeDtypeStruct((tokens, _WIDTH), jnp.float32),
    )(acts, nodes, w_out)
    return out.reshape(orig_shape)


# speedup vs baseline: 4.0613x; 2.3409x over previous
import functools
import math

import jax
import jax.numpy as jnp
from jax import lax
from jax.experimental import pallas as pl
from jax.experimental.pallas import tpu as pltpu
from jax.experimental.pallas import tpu_sc as plsc

_DEPTH = 11
_NLEVELS = _DEPTH + 1
_N_NODES = 2 ** _NLEVELS - 1
_WIDTH = 2048
_TOKENS = 8192
_TILE = 256
_NTILES = _TOKENS // _TILE
_SHALLOW = 8

_NW = 32
_BPW = _TOKENS // _NW
_CHUNK = 16
_NCH = _BPW // _CHUNK


def _gelu(x):
    return 0.5 * x * (1.0 + lax.erf(x * (1.0 / math.sqrt(2.0))))


def _shallow_kernel(x_ref, w_ref, b_ref, acts_ref, n8_ref):
    x = x_ref[...]
    t = x.shape[0]
    L = lax.dot_general(
        x, w_ref[...], (((1,), (1,)), ((), ())),
        precision=lax.Precision.HIGHEST,
        preferred_element_type=jnp.float32)
    bias_row = b_ref[0:1, :]
    ids = lax.broadcasted_iota(jnp.int32, (t, 256), 1)
    lane = lax.broadcasted_iota(jnp.int32, (t, _SHALLOW), 1)
    n = jnp.zeros((t, 1), jnp.int32)
    acts = jnp.zeros((t, _SHALLOW), jnp.float32)
    for d in range(_SHALLOW):
        sel = ids == n
        logit = jnp.sum(jnp.where(sel, L, 0.0), axis=1, keepdims=True)
        logit = logit + jnp.sum(
            jnp.where(sel, jnp.broadcast_to(bias_row, (t, 256)), 0.0),
            axis=1, keepdims=True)
        act = _gelu(logit)
        acts = jnp.where(lane == d, act, acts)
        n = 2 * n + 1 + (logit > 0.0).astype(jnp.int32)
    acts_ref[...] = acts
    n8_ref[...] = n


def _sc_gather_body(w_hbm, idx_hbm, rows_hbm, idx_v, buf0, buf1,
                    sem0, sem1):
    wid = lax.axis_index("s") * 2 + lax.axis_index("c")
    base = wid * _BPW
    pltpu.sync_copy(idx_hbm.at[wid], idx_v)
    bufs = (buf0, buf1)
    sems = (sem0, sem1)
    cps = [None, None]
    cps[0] = pltpu.async_copy(w_hbm.at[idx_v.at[0]], buf0, sem0)
    for c in range(_NCH):
        if c + 1 < _NCH:
            cps[(c + 1) % 2] = pltpu.async_copy(
                w_hbm.at[idx_v.at[c + 1]], bufs[(c + 1) % 2],
                sems[(c + 1) % 2])
        cps[c % 2].wait()
        pltpu.sync_copy(bufs[c % 2],
                        rows_hbm.at[pl.ds(base + c * _CHUNK, _CHUNK)])


@functools.cache
def _make_sc_gather():
    return functools.partial(
        pl.kernel,
        mesh=plsc.VectorSubcoreMesh(core_axis_name="c",
                                    subcore_axis_name="s"),
        out_type=jax.ShapeDtypeStruct((_TOKENS, _WIDTH), jnp.float32),
        scratch_types=[
            pltpu.VMEM((_NCH, _CHUNK), jnp.int32),
            pltpu.VMEM((_CHUNK, _WIDTH), jnp.float32),
            pltpu.VMEM((_CHUNK, _WIDTH), jnp.float32),
            pltpu.SemaphoreType.DMA,
            pltpu.SemaphoreType.DMA,
        ],
    )(_sc_gather_body)


def _gather_rows(w, idx):
    return _make_sc_gather()(w, idx)


def _dot_kernel(x_ref, rows_ref, n_ref, b_ref, act_ref, nnext_ref, *,
                level):
    x = x_ref[...]
    r = rows_ref[...]
    t = x.shape[0]
    start = 2 ** level - 1
    size = 2 ** level
    n = n_ref[...]
    local = n - start
    ids = lax.broadcasted_iota(jnp.int32, (t, size), 1)
    sel = ids == local
    brow = b_ref[level:level + 1, :size]
    bsum = jnp.sum(jnp.where(sel, jnp.broadcast_to(brow, (t, size)), 0.0),
                   axis=1, keepdims=True)
    logit = jnp.sum(x * r, axis=1, keepdims=True) + bsum
    act_ref[...] = _gelu(logit)
    nnext_ref[...] = 2 * n + 1 + (logit > 0.0).astype(jnp.int32)


def _out_kernel(leaf_ref, acts_sh_ref, a8_ref, a9_ref, a10_ref, a11_ref,
                wout_ref, out_ref):
    leaf1 = leaf_ref[...] + 1
    t = leaf1.shape[0]
    cols = lax.broadcasted_iota(jnp.int32, (t, _N_NODES + 1), 1)
    deep = (a8_ref, a9_ref, a10_ref, a11_ref)
    a = jnp.zeros((t, _N_NODES + 1), jnp.float32)
    for d in range(_NLEVELS):
        nd = (leaf1 >> (_DEPTH - d)) - 1
        if d < _SHALLOW:
            act_d = acts_sh_ref[:, d:d + 1]
        else:
            act_d = deep[d - _SHALLOW][...]
        a = jnp.where(cols == nd, act_d, a)
    out_ref[...] = lax.dot_general(
        a.astype(jnp.bfloat16), wout_ref[...],
        (((1,), (0,)), ((), ())),
        preferred_element_type=jnp.float32)


def kernel(input, in_weight, in_bias, out_weight):
    orig_shape = input.shape
    x = input.reshape(-1, _WIDTH)

    bias_sh = jnp.zeros((8, 256), jnp.float32)
    bias_sh = bias_sh.at[0, :255].set(in_bias[:255])
    bias_lvl = jnp.zeros((_NLEVELS + 4, _WIDTH), jnp.float32)
    for d in range(_SHALLOW, _NLEVELS):
        row = jnp.zeros((_WIDTH,), jnp.float32)
        row = lax.dynamic_update_slice(
            row, in_bias[2 ** d - 1: 2 ** (d + 1) - 1], (0,))
        bias_lvl = bias_lvl.at[d].set(row)
    wout_p = jnp.pad(out_weight.astype(jnp.bfloat16), ((0, 1), (0, 0)))

    acts_sh, n8 = pl.pallas_call(
        _shallow_kernel,
        grid=(_NTILES,),
        in_specs=[
            pl.BlockSpec((_TILE, _WIDTH), lambda i: (i, 0)),
            pl.BlockSpec((256, _WIDTH), lambda i: (0, 0)),
            pl.BlockSpec((8, 256), lambda i: (0, 0)),
        ],
        out_specs=[
            pl.BlockSpec((_TILE, _SHALLOW), lambda i: (i, 0)),
            pl.BlockSpec((_TILE, 1), lambda i: (i, 0)),
        ],
        out_shape=[
            jax.ShapeDtypeStruct((_TOKENS, _SHALLOW), jnp.float32),
            jax.ShapeDtypeStruct((_TOKENS, 1), jnp.int32),
        ],
    )(x, in_weight, bias_sh)

    n = n8
    acts_deep = []
    leaf = None
    for d in range(_SHALLOW, _NLEVELS):
        idx = n.reshape(_NW, _NCH, _CHUNK)
        rows = _gather_rows(in_weight, idx)
        if d == _DEPTH:
            leaf = n
        act_d, nnext = pl.pallas_call(
            functools.partial(_dot_kernel, level=d),
            grid=(_NTILES,),
            in_specs=[
                pl.BlockSpec((_TILE, _WIDTH), lambda i: (i, 0)),
                pl.BlockSpec((_TILE, _WIDTH), lambda i: (i, 0)),
                pl.BlockSpec((_TILE, 1), lambda i: (i, 0)),
                pl.BlockSpec((_NLEVELS + 4, _WIDTH), lambda i: (0, 0)),
            ],
            out_specs=[
                pl.BlockSpec((_TILE, 1), lambda i: (i, 0)),
                pl.BlockSpec((_TILE, 1), lambda i: (i, 0)),
            ],
            out_shape=[
                jax.ShapeDtypeStruct((_TOKENS, 1), jnp.float32),
                jax.ShapeDtypeStruct((_TOKENS, 1), jnp.int32),
            ],
        )(x, rows, n, bias_lvl)
        acts_deep.append(act_d)
        n = nnext

    out = pl.pallas_call(
        _out_kernel,
        grid=(_NTILES,),
        in_specs=[
            pl.BlockSpec((_TILE, 1), lambda i: (i, 0)),
            pl.BlockSpec((_TILE, _SHALLOW), lambda i: (i, 0)),
            pl.BlockSpec((_TILE, 1), lambda i: (i, 0)),
            pl.BlockSpec((_TILE, 1), lambda i: (i, 0)),
            pl.BlockSpec((_TILE, 1), lambda i: (i, 0)),
            pl.BlockSpec((_TILE, 1), lambda i: (i, 0)),
            pl.BlockSpec((_N_NODES + 1, _WIDTH), lambda i: (0, 0)),
        ],
        out_specs=pl.BlockSpec((_TILE, _WIDTH), lambda i: (i, 0)),
        out_shape=jax.ShapeDtypeStruct((_TOKENS, _WIDTH), jnp.float32),
    )(leaf, acts_sh, *acts_deep, wout_p)
    return out.reshape(orig_shape)
